# baseline (speedup 1.0000x reference)
import functools

import jax
import jax.numpy as jnp
from jax import lax
from jax.experimental import pallas as pl
from jax.experimental.pallas import tpu as pltpu
from jax.experimental.pallas import tpu_sc as plsc

N = 10000
E = 320000
NFEAT = 128
NHID = 128
NCLASS = 64
NHEAD = 4
DPH = NHID // NHEAD

NC = 2
NS = 16
L = 16
NW = NC * NS
EW = E // NW
C = 80
NCHUNK = EW // C
RPT = N // NS
ZROWS = 125


def _zero_rows(ref, nrows, rw):
    z = jnp.zeros((L,), jnp.float32)

    def body(i, _):
        for k in range(rw // L):
            ref[i, pl.ds(k * L, L)] = z
        if rw % L:
            ref[i, pl.ds(rw - L, L)] = z
        return 0

    lax.fori_loop(0, nrows, body, 0)


def _make_edge_kernel(d, nh, sw, rw, dst_off=None):
    mesh = plsc.VectorSubcoreMesh(
        core_axis_name="c", subcore_axis_name="s", num_cores=NC,
        num_subcores=NS)
    doff = nh if dst_off is None else dst_off

    @functools.partial(
        pl.kernel,
        out_type=jax.ShapeDtypeStruct((NC, N, rw), jnp.float32),
        mesh=mesh,
        compiler_params=pltpu.CompilerParams(
            use_tc_tiling_on_sc=False, needs_layout_passes=False),
        scratch_types=[
            pltpu.VMEM_SHARED((N, rw), jnp.float32),
            pltpu.VMEM((ZROWS, rw), jnp.float32),
            pltpu.VMEM((C,), jnp.int32),
            pltpu.VMEM((C,), jnp.int32),
            pltpu.VMEM((C, sw), jnp.float32),
            pltpu.VMEM((C, sw), jnp.float32),
            pltpu.VMEM((C, d), jnp.float32),
            pltpu.VMEM((C, rw), jnp.float32),
        ],
    )
    def edge_kernel(src_h, dst_h, feat_h, sc_h, out_h,
                    acc, zbuf, idx_s, idx_d, ss, sd, hrows, scaled):
        cid = lax.axis_index("c")
        sid = lax.axis_index("s")

        _zero_rows(zbuf, ZROWS, rw)
        for j in range(RPT // ZROWS):
            pltpu.sync_copy(
                zbuf, acc.at[pl.ds(sid * RPT + j * ZROWS, ZROWS)])
        zt = jnp.zeros((L,), jnp.float32)

        def ztail(i, _):
            scaled[i, pl.ds(rw - L, L)] = zt
            return 0

        lax.fori_loop(0, C, ztail, 0)
        plsc.subcore_barrier()

        base0 = (cid * NS + sid) * EW
        ridx0 = lax.iota(jnp.int32, L)

        def chunk(k, _):
            base = base0 + k * C
            pltpu.sync_copy(src_h.at[pl.ds(base, C)], idx_s)
            pltpu.sync_copy(dst_h.at[pl.ds(base, C)], idx_d)
            pltpu.sync_copy(sc_h.at[idx_s], ss)
            pltpu.sync_copy(sc_h.at[idx_d], sd)
            pltpu.sync_copy(feat_h.at[idx_d], hrows)

            for g in range(C // L):
                ridx = ridx0 + (g * L)
                ws = []
                for h in range(nh):
                    ch_s = jnp.full((L,), h, jnp.int32)
                    ch_d = jnp.full((L,), doff + h, jnp.int32)
                    e = (plsc.load_gather(ss, [ridx, ch_s])
                         + plsc.load_gather(sd, [ridx, ch_d]))
                    e = jnp.maximum(e, 0.2 * e)
                    w = jnp.exp(e)
                    ws.append(w)
                    plsc.store_scatter(
                        scaled,
                        [ridx, jnp.full((L,), d + h, jnp.int32)], w)
                for c in range(d):
                    cc = jnp.full((L,), c, jnp.int32)
                    v = plsc.load_gather(hrows, [ridx, cc])
                    plsc.store_scatter(scaled, [ridx, cc],
                                       v * ws[c // (d // nh)])

            pltpu.sync_copy(scaled, acc.at[idx_s], add=True)
            return 0

        lax.fori_loop(0, NCHUNK, chunk, 0)

        plsc.subcore_barrier()
        pltpu.sync_copy(acc.at[pl.ds(sid * RPT, RPT)],
                        out_h.at[cid, pl.ds(sid * RPT, RPT)])

    return edge_kernel


_edge1 = _make_edge_kernel(d=NHID, nh=NHEAD, sw=2 * NHEAD, rw=NHID + 2 * NHEAD)
_edge2 = _make_edge_kernel(d=NCLASS, nh=1, sw=8, rw=NCLASS + 8, dst_off=4)

_BLK = 400
_GRID = N // _BLK


def _dense1_body(x_ref, w_ref, a_ref, h_ref, s_ref):
    h = jnp.dot(x_ref[...], w_ref[...], preferred_element_type=jnp.float32)
    h_ref[...] = h
    s_ref[...] = jnp.dot(h, a_ref[...], preferred_element_type=jnp.float32)


def _dense1(x, w0cat, a8):
    return pl.pallas_call(
        _dense1_body,
        grid=(_GRID,),
        in_specs=[
            pl.BlockSpec((_BLK, NFEAT), lambda i: (i, 0)),
            pl.BlockSpec((NFEAT, NHID), lambda i: (0, 0)),
            pl.BlockSpec((NHID, 2 * NHEAD), lambda i: (0, 0)),
        ],
        out_specs=[
            pl.BlockSpec((_BLK, NHID), lambda i: (i, 0)),
            pl.BlockSpec((_BLK, 2 * NHEAD), lambda i: (i, 0)),
        ],
        out_shape=[
            jax.ShapeDtypeStruct((N, NHID), jnp.float32),
            jax.ShapeDtypeStruct((N, 2 * NHEAD), jnp.float32),
        ],
    )(x, w0cat, a8)


def _dense2_body(p_ref, w_ref, a_ref, g_ref, t_ref):
    p = p_ref[...]
    n = p[0] + p[1]
    cols = []
    for h in range(NHEAD):
        den = n[:, NHID + h:NHID + h + 1] + 1e-16
        cols.append(n[:, h * DPH:(h + 1) * DPH] / den)
    h2 = jnp.concatenate(cols, axis=1)
    h2 = jnp.where(h2 > 0, h2, jnp.exp(jnp.minimum(h2, 0.0)) - 1.0)
    g = jnp.dot(h2, w_ref[...], preferred_element_type=jnp.float32)
    g_ref[...] = g
    t_ref[...] = jnp.dot(g, a_ref[...], preferred_element_type=jnp.float32)


def _dense2(p1, w1, a2):
    rw = NHID + 2 * NHEAD
    return pl.pallas_call(
        _dense2_body,
        grid=(_GRID,),
        in_specs=[
            pl.BlockSpec((NC, _BLK, rw), lambda i: (0, i, 0)),
            pl.BlockSpec((NHID, NCLASS), lambda i: (0, 0)),
            pl.BlockSpec((NCLASS, 8), lambda i: (0, 0)),
        ],
        out_specs=[
            pl.BlockSpec((_BLK, NCLASS), lambda i: (i, 0)),
            pl.BlockSpec((_BLK, 8), lambda i: (i, 0)),
        ],
        out_shape=[
            jax.ShapeDtypeStruct((N, NCLASS), jnp.float32),
            jax.ShapeDtypeStruct((N, 8), jnp.float32),
        ],
    )(p1, w1, a2)


def _combine_body(p_ref, o_ref):
    p = p_ref[...]
    n = p[0] + p[1]
    o_ref[...] = n[:, :NCLASS] / (n[:, NCLASS:NCLASS + 1] + 1e-16)


def _combine(p2):
    rw = NCLASS + 8
    return pl.pallas_call(
        _combine_body,
        grid=(_GRID,),
        in_specs=[pl.BlockSpec((NC, _BLK, rw), lambda i: (0, i, 0))],
        out_specs=pl.BlockSpec((_BLK, NCLASS), lambda i: (i, 0)),
        out_shape=jax.ShapeDtypeStruct((N, NCLASS), jnp.float32),
    )(p2)


def kernel(x, adj, W0, a0, W1, a1):
    src = adj[0]
    dst = adj[1]
    w0cat = jnp.transpose(W0, (1, 0, 2)).reshape(NFEAT, NHID)
    a8 = jnp.zeros((NHID, 2 * NHEAD), jnp.float32)
    for h in range(NHEAD):
        a8 = a8.at[h * DPH:(h + 1) * DPH, h].set(a0[h, :DPH])
        a8 = a8.at[h * DPH:(h + 1) * DPH, NHEAD + h].set(a0[h, DPH:])
    a2 = jnp.zeros((NCLASS, 8), jnp.float32)
    a2 = a2.at[:, 0].set(a1[:NCLASS]).at[:, 4].set(a1[NCLASS:])

    h, s8 = _dense1(x, w0cat, a8)
    p1 = _edge1(src, dst, h, s8)
    g, t = _dense2(p1, W1, a2)
    p2 = _edge2(src, dst, g, t)
    return _combine(p2)


# speedup vs baseline: 1.1161x; 1.1161x over previous
import functools

import jax
import jax.numpy as jnp
from jax import lax
from jax.experimental import pallas as pl
from jax.experimental.pallas import tpu as pltpu
from jax.experimental.pallas import tpu_sc as plsc

N = 10000
E = 320000
NFEAT = 128
NHID = 128
NCLASS = 64
NHEAD = 4
DPH = NHID // NHEAD

NC = 2
NS = 16
L = 16
NW = NC * NS
EW = E // NW
C = 80
NCHUNK = EW // C
RPT = N // NS
ZROWS = 125


def _zero_rows(ref, nrows, rw):
    z = jnp.zeros((L,), jnp.float32)

    def body(i, _):
        for k in range(rw // L):
            ref[i, pl.ds(k * L, L)] = z
        if rw % L:
            ref[i, pl.ds(rw - L, L)] = z
        return 0

    lax.fori_loop(0, nrows, body, 0)


def _make_edge_kernel(d, nh, sw, rw, dst_off=None):
    mesh = plsc.VectorSubcoreMesh(
        core_axis_name="c", subcore_axis_name="s", num_cores=NC,
        num_subcores=NS)
    doff = nh if dst_off is None else dst_off

    @functools.partial(
        pl.kernel,
        out_type=jax.ShapeDtypeStruct((NC, N, rw), jnp.float32),
        mesh=mesh,
        compiler_params=pltpu.CompilerParams(
            use_tc_tiling_on_sc=False, needs_layout_passes=False),
        scratch_types=[
            pltpu.VMEM_SHARED((N, rw), jnp.float32),
            pltpu.VMEM((ZROWS, rw), jnp.float32),
            pltpu.VMEM((C,), jnp.int32),
            pltpu.VMEM((C,), jnp.int32),
            pltpu.VMEM((C, sw), jnp.float32),
            pltpu.VMEM((C, sw), jnp.float32),
            pltpu.VMEM((C, d), jnp.float32),
            pltpu.VMEM((C, rw), jnp.float32),
            pltpu.SemaphoreType.DMA,
        ],
    )
    def edge_kernel(src_h, dst_h, feat_h, sc_h, out_h,
                    acc, zbuf, idx_s, idx_d, ss, sd, hrows, scaled, gsem):
        cid = lax.axis_index("c")
        sid = lax.axis_index("s")

        _zero_rows(zbuf, ZROWS, rw)
        for j in range(RPT // ZROWS):
            pltpu.sync_copy(
                zbuf, acc.at[pl.ds(sid * RPT + j * ZROWS, ZROWS)])
        zt = jnp.zeros((L,), jnp.float32)

        def ztail(i, _):
            scaled[i, pl.ds(rw - L, L)] = zt
            return 0

        lax.fori_loop(0, C, ztail, 0)
        plsc.subcore_barrier()

        base0 = (cid * NS + sid) * EW
        ridx0 = lax.iota(jnp.int32, L)

        def chunk(k, _):
            base = base0 + k * C
            pltpu.sync_copy(src_h.at[pl.ds(base, C)], idx_s)
            pltpu.sync_copy(dst_h.at[pl.ds(base, C)], idx_d)
            pltpu.async_copy(sc_h.at[idx_s], ss, gsem)
            pltpu.async_copy(sc_h.at[idx_d], sd, gsem)
            pltpu.async_copy(feat_h.at[idx_d], hrows, gsem)
            pltpu.make_async_copy(sc_h.at[idx_s], ss, gsem).wait()
            pltpu.make_async_copy(sc_h.at[idx_d], sd, gsem).wait()
            pltpu.make_async_copy(feat_h.at[idx_d], hrows, gsem).wait()

            for g in range(C // L):
                ridx = ridx0 + (g * L)
                ws = []
                for h in range(nh):
                    ch_s = jnp.full((L,), h, jnp.int32)
                    ch_d = jnp.full((L,), doff + h, jnp.int32)
                    e = (plsc.load_gather(ss, [ridx, ch_s])
                         + plsc.load_gather(sd, [ridx, ch_d]))
                    e = jnp.maximum(e, 0.2 * e)
                    w = jnp.exp(e)
                    ws.append(w)
                    plsc.store_scatter(
                        scaled,
                        [ridx, jnp.full((L,), d + h, jnp.int32)], w)
                for c in range(d):
                    cc = jnp.full((L,), c, jnp.int32)
                    v = plsc.load_gather(hrows, [ridx, cc])
                    plsc.store_scatter(scaled, [ridx, cc],
                                       v * ws[c // (d // nh)])

            pltpu.sync_copy(scaled, acc.at[idx_s], add=True)
            return 0

        lax.fori_loop(0, NCHUNK, chunk, 0)

        plsc.subcore_barrier()
        pltpu.sync_copy(acc.at[pl.ds(sid * RPT, RPT)],
                        out_h.at[cid, pl.ds(sid * RPT, RPT)])

    return edge_kernel


_edge1 = _make_edge_kernel(d=NHID, nh=NHEAD, sw=2 * NHEAD, rw=NHID + 2 * NHEAD)
_edge2 = _make_edge_kernel(d=NCLASS, nh=1, sw=8, rw=NCLASS + 8, dst_off=4)

_BLK = 400
_GRID = N // _BLK


def _dense1_body(x_ref, w_ref, a_ref, h_ref, s_ref):
    h = jnp.dot(x_ref[...], w_ref[...], preferred_element_type=jnp.float32)
    h_ref[...] = h
    s_ref[...] = jnp.dot(h, a_ref[...], preferred_element_type=jnp.float32)


def _dense1(x, w0cat, a8):
    return pl.pallas_call(
        _dense1_body,
        grid=(_GRID,),
        in_specs=[
            pl.BlockSpec((_BLK, NFEAT), lambda i: (i, 0)),
            pl.BlockSpec((NFEAT, NHID), lambda i: (0, 0)),
            pl.BlockSpec((NHID, 2 * NHEAD), lambda i: (0, 0)),
        ],
        out_specs=[
            pl.BlockSpec((_BLK, NHID), lambda i: (i, 0)),
            pl.BlockSpec((_BLK, 2 * NHEAD), lambda i: (i, 0)),
        ],
        out_shape=[
            jax.ShapeDtypeStruct((N, NHID), jnp.float32),
            jax.ShapeDtypeStruct((N, 2 * NHEAD), jnp.float32),
        ],
    )(x, w0cat, a8)


def _dense2_body(p_ref, w_ref, a_ref, g_ref, t_ref):
    p = p_ref[...]
    n = p[0] + p[1]
    cols = []
    for h in range(NHEAD):
        den = n[:, NHID + h:NHID + h + 1] + 1e-16
        cols.append(n[:, h * DPH:(h + 1) * DPH] / den)
    h2 = jnp.concatenate(cols, axis=1)
    h2 = jnp.where(h2 > 0, h2, jnp.exp(jnp.minimum(h2, 0.0)) - 1.0)
    g = jnp.dot(h2, w_ref[...], preferred_element_type=jnp.float32)
    g_ref[...] = g
    t_ref[...] = jnp.dot(g, a_ref[...], preferred_element_type=jnp.float32)


def _dense2(p1, w1, a2):
    rw = NHID + 2 * NHEAD
    return pl.pallas_call(
        _dense2_body,
        grid=(_GRID,),
        in_specs=[
            pl.BlockSpec((NC, _BLK, rw), lambda i: (0, i, 0)),
            pl.BlockSpec((NHID, NCLASS), lambda i: (0, 0)),
            pl.BlockSpec((NCLASS, 8), lambda i: (0, 0)),
        ],
        out_specs=[
            pl.BlockSpec((_BLK, NCLASS), lambda i: (i, 0)),
            pl.BlockSpec((_BLK, 8), lambda i: (i, 0)),
        ],
        out_shape=[
            jax.ShapeDtypeStruct((N, NCLASS), jnp.float32),
            jax.ShapeDtypeStruct((N, 8), jnp.float32),
        ],
    )(p1, w1, a2)


def _combine_body(p_ref, o_ref):
    p = p_ref[...]
    n = p[0] + p[1]
    o_ref[...] = n[:, :NCLASS] / (n[:, NCLASS:NCLASS + 1] + 1e-16)


def _combine(p2):
    rw = NCLASS + 8
    return pl.pallas_call(
        _combine_body,
        grid=(_GRID,),
        in_specs=[pl.BlockSpec((NC, _BLK, rw), lambda i: (0, i, 0))],
        out_specs=pl.BlockSpec((_BLK, NCLASS), lambda i: (i, 0)),
        out_shape=jax.ShapeDtypeStruct((N, NCLASS), jnp.float32),
    )(p2)


def kernel(x, adj, W0, a0, W1, a1):
    src = adj[0]
    dst = adj[1]
    w0cat = jnp.transpose(W0, (1, 0, 2)).reshape(NFEAT, NHID)
    a8 = jnp.zeros((NHID, 2 * NHEAD), jnp.float32)
    for h in range(NHEAD):
        a8 = a8.at[h * DPH:(h + 1) * DPH, h].set(a0[h, :DPH])
        a8 = a8.at[h * DPH:(h + 1) * DPH, NHEAD + h].set(a0[h, DPH:])
    a2 = jnp.zeros((NCLASS, 8), jnp.float32)
    a2 = a2.at[:, 0].set(a1[:NCLASS]).at[:, 4].set(a1[NCLASS:])

    h, s8 = _dense1(x, w0cat, a8)
    p1 = _edge1(src, dst, h, s8)
    g, t = _dense2(p1, W1, a2)
    p2 = _edge2(src, dst, g, t)
    return _combine(p2)


# speedup vs baseline: 1.1619x; 1.0410x over previous
import functools

import jax
import jax.numpy as jnp
from jax import lax
from jax.experimental import pallas as pl
from jax.experimental.pallas import tpu as pltpu
from jax.experimental.pallas import tpu_sc as plsc

N = 10000
E = 320000
NFEAT = 128
NHID = 128
NCLASS = 64
NHEAD = 4
DPH = NHID // NHEAD

NC = 2
NS = 16
L = 16
NW = NC * NS
EW = E // NW
C = 80
NCHUNK = EW // C
RPT = N // NS
ZROWS = 125


def _zero_rows(ref, nrows, rw):
    z = jnp.zeros((L,), jnp.float32)

    def body(i, _):
        for k in range(rw // L):
            ref[i, pl.ds(k * L, L)] = z
        if rw % L:
            ref[i, pl.ds(rw - L, L)] = z
        return 0

    lax.fori_loop(0, nrows, body, 0)


def _make_edge_kernel(d, nh, sw, rw, dst_off=None):
    mesh = plsc.VectorSubcoreMesh(
        core_axis_name="c", subcore_axis_name="s", num_cores=NC,
        num_subcores=NS)
    doff = nh if dst_off is None else dst_off

    @functools.partial(
        pl.kernel,
        out_type=jax.ShapeDtypeStruct((NC, N, rw), jnp.float32),
        mesh=mesh,
        compiler_params=pltpu.CompilerParams(
            use_tc_tiling_on_sc=False, needs_layout_passes=False),
        scratch_types=[
            pltpu.VMEM_SHARED((N, rw), jnp.float32),
            pltpu.VMEM((ZROWS, rw), jnp.float32),
            pltpu.VMEM((C,), jnp.int32),
            pltpu.VMEM((C,), jnp.int32),
            pltpu.VMEM((C, sw), jnp.float32),
            pltpu.VMEM((C, sw), jnp.float32),
            pltpu.VMEM((C, d), jnp.float32),
            pltpu.VMEM((C, rw), jnp.float32),
            pltpu.SemaphoreType.DMA,
        ],
    )
    def edge_kernel(src_h, dst_h, feat_h, sc_h, out_h,
                    acc, zbuf, idx_s, idx_d, ss, sd, hrows, scaled, gsem):
        cid = lax.axis_index("c")
        sid = lax.axis_index("s")

        _zero_rows(zbuf, ZROWS, rw)
        for j in range(RPT // ZROWS):
            pltpu.sync_copy(
                zbuf, acc.at[pl.ds(sid * RPT + j * ZROWS, ZROWS)])
        zt = jnp.zeros((L,), jnp.float32)

        def ztail(i, _):
            scaled[i, pl.ds(rw - L, L)] = zt
            return 0

        lax.fori_loop(0, C, ztail, 0)
        plsc.subcore_barrier()

        base0 = (cid * NS + sid) * EW
        ridx0 = lax.iota(jnp.int32, L)

        def chunk(k, _):
            base = base0 + k * C
            pltpu.async_copy(src_h.at[pl.ds(base, C)], idx_s, gsem)
            pltpu.async_copy(dst_h.at[pl.ds(base, C)], idx_d, gsem)
            pltpu.make_async_copy(src_h.at[pl.ds(base, C)], idx_s,
                                  gsem).wait()
            pltpu.make_async_copy(dst_h.at[pl.ds(base, C)], idx_d,
                                  gsem).wait()
            pltpu.async_copy(sc_h.at[idx_s], ss, gsem)
            pltpu.async_copy(sc_h.at[idx_d], sd, gsem)
            pltpu.async_copy(feat_h.at[idx_d], hrows, gsem)
            pltpu.make_async_copy(sc_h.at[idx_s], ss, gsem).wait()
            pltpu.make_async_copy(sc_h.at[idx_d], sd, gsem).wait()
            pltpu.make_async_copy(feat_h.at[idx_d], hrows, gsem).wait()

            for g in range(C // L):
                ridx = ridx0 + (g * L)
                ws = []
                for h in range(nh):
                    ch_s = jnp.full((L,), h, jnp.int32)
                    ch_d = jnp.full((L,), doff + h, jnp.int32)
                    e = (plsc.load_gather(ss, [ridx, ch_s])
                         + plsc.load_gather(sd, [ridx, ch_d]))
                    e = jnp.maximum(e, 0.2 * e)
                    w = jnp.exp(e)
                    ws.append(w)
                    plsc.store_scatter(
                        scaled,
                        [ridx, jnp.full((L,), d + h, jnp.int32)], w)
                for c in range(d):
                    cc = jnp.full((L,), c, jnp.int32)
                    v = plsc.load_gather(hrows, [ridx, cc])
                    plsc.store_scatter(scaled, [ridx, cc],
                                       v * ws[c // (d // nh)])

            pltpu.sync_copy(scaled, acc.at[idx_s], add=True)
            return 0

        lax.fori_loop(0, NCHUNK, chunk, 0)

        plsc.subcore_barrier()
        pltpu.sync_copy(acc.at[pl.ds(sid * RPT, RPT)],
                        out_h.at[cid, pl.ds(sid * RPT, RPT)])

    return edge_kernel


_edge1 = _make_edge_kernel(d=NHID, nh=NHEAD, sw=2 * NHEAD, rw=NHID + 2 * NHEAD)
_edge2 = _make_edge_kernel(d=NCLASS, nh=1, sw=8, rw=NCLASS + 8, dst_off=4)

_BLK = 400
_GRID = N // _BLK


def _dense1_body(x_ref, w_ref, a_ref, h_ref, s_ref):
    h = jnp.dot(x_ref[...], w_ref[...], preferred_element_type=jnp.float32)
    h_ref[...] = h
    s_ref[...] = jnp.dot(h, a_ref[...], preferred_element_type=jnp.float32)


def _dense1(x, w0cat, a8):
    return pl.pallas_call(
        _dense1_body,
        grid=(_GRID,),
        in_specs=[
            pl.BlockSpec((_BLK, NFEAT), lambda i: (i, 0)),
            pl.BlockSpec((NFEAT, NHID), lambda i: (0, 0)),
            pl.BlockSpec((NHID, 2 * NHEAD), lambda i: (0, 0)),
        ],
        out_specs=[
            pl.BlockSpec((_BLK, NHID), lambda i: (i, 0)),
            pl.BlockSpec((_BLK, 2 * NHEAD), lambda i: (i, 0)),
        ],
        out_shape=[
            jax.ShapeDtypeStruct((N, NHID), jnp.float32),
            jax.ShapeDtypeStruct((N, 2 * NHEAD), jnp.float32),
        ],
    )(x, w0cat, a8)


def _dense2_body(p_ref, w_ref, a_ref, g_ref, t_ref):
    p = p_ref[...]
    n = p[0] + p[1]
    cols = []
    for h in range(NHEAD):
        den = n[:, NHID + h:NHID + h + 1] + 1e-16
        cols.append(n[:, h * DPH:(h + 1) * DPH] / den)
    h2 = jnp.concatenate(cols, axis=1)
    h2 = jnp.where(h2 > 0, h2, jnp.exp(jnp.minimum(h2, 0.0)) - 1.0)
    g = jnp.dot(h2, w_ref[...], preferred_element_type=jnp.float32)
    g_ref[...] = g
    t_ref[...] = jnp.dot(g, a_ref[...], preferred_element_type=jnp.float32)


def _dense2(p1, w1, a2):
    rw = NHID + 2 * NHEAD
    return pl.pallas_call(
        _dense2_body,
        grid=(_GRID,),
        in_specs=[
            pl.BlockSpec((NC, _BLK, rw), lambda i: (0, i, 0)),
            pl.BlockSpec((NHID, NCLASS), lambda i: (0, 0)),
            pl.BlockSpec((NCLASS, 8), lambda i: (0, 0)),
        ],
        out_specs=[
            pl.BlockSpec((_BLK, NCLASS), lambda i: (i, 0)),
            pl.BlockSpec((_BLK, 8), lambda i: (i, 0)),
        ],
        out_shape=[
            jax.ShapeDtypeStruct((N, NCLASS), jnp.float32),
            jax.ShapeDtypeStruct((N, 8), jnp.float32),
        ],
    )(p1, w1, a2)


def _combine_body(p_ref, o_ref):
    p = p_ref[...]
    n = p[0] + p[1]
    o_ref[...] = n[:, :NCLASS] / (n[:, NCLASS:NCLASS + 1] + 1e-16)


def _combine(p2):
    rw = NCLASS + 8
    return pl.pallas_call(
        _combine_body,
        grid=(_GRID,),
        in_specs=[pl.BlockSpec((NC, _BLK, rw), lambda i: (0, i, 0))],
        out_specs=pl.BlockSpec((_BLK, NCLASS), lambda i: (i, 0)),
        out_shape=jax.ShapeDtypeStruct((N, NCLASS), jnp.float32),
    )(p2)


def kernel(x, adj, W0, a0, W1, a1):
    src = adj[0]
    dst = adj[1]
    w0cat = jnp.transpose(W0, (1, 0, 2)).reshape(NFEAT, NHID)
    a8 = jnp.zeros((NHID, 2 * NHEAD), jnp.float32)
    for h in range(NHEAD):
        a8 = a8.at[h * DPH:(h + 1) * DPH, h].set(a0[h, :DPH])
        a8 = a8.at[h * DPH:(h + 1) * DPH, NHEAD + h].set(a0[h, DPH:])
    a2 = jnp.zeros((NCLASS, 8), jnp.float32)
    a2 = a2.at[:, 0].set(a1[:NCLASS]).at[:, 4].set(a1[NCLASS:])

    h, s8 = _dense1(x, w0cat, a8)
    p1 = _edge1(src, dst, h, s8)
    g, t = _dense2(p1, W1, a2)
    p2 = _edge2(src, dst, g, t)
    return _combine(p2)
